# baseline (speedup 1.0000x reference)
import functools

import jax
import jax.numpy as jnp
from jax import lax
from jax.experimental import pallas as pl
from jax.experimental.pallas import tpu as pltpu
from jax.experimental.pallas import tpu_sc as plsc

EMB = 32
BATCH = 16384
HIST = 200
NC, NS = 2, 16
NW = NC * NS
BT = BATCH // 128
BT_PER_W = BT // NW
CW = BT_PER_W * 128
TW = BT_PER_W * 128 * EMB
HROW = BT * 128 * EMB

_mesh = plsc.VectorSubcoreMesh(core_axis_name="c", subcore_axis_name="s")


@functools.partial(
    pl.kernel,
    mesh=_mesh,
    out_type=jax.ShapeDtypeStruct((HIST, HROW), jnp.float32),
    scratch_types=[
        pltpu.VMEM((2, CW), jnp.int32),
        pltpu.VMEM((2, CW, EMB), jnp.float32),
        pltpu.VMEM((2, TW), jnp.float32),
        pltpu.SemaphoreType.DMA,
        pltpu.SemaphoreType.DMA,
        pltpu.SemaphoreType.DMA,
        pltpu.SemaphoreType.DMA,
        pltpu.SemaphoreType.DMA,
        pltpu.SemaphoreType.DMA,
    ],
    compiler_params=pltpu.CompilerParams(use_tc_tiling_on_sc=False,
                                         needs_layout_passes=False),
)
def _emb_lookup(idx_hbm, table_hbm, out_hbm, idx_v, g_v, t_v,
                isem0, isem1, gsem0, gsem1, osem0, osem1):
    wid = lax.axis_index("s") * NC + lax.axis_index("c")
    col0 = wid * CW
    bt0 = wid * BT_PER_W
    isem = (isem0, isem1)
    gsem = (gsem0, gsem1)
    osem = (osem0, osem1)
    iota16 = jnp.arange(16, dtype=jnp.int32)
    pat_lo = (iota16 // 8) * 4096 + (iota16 % 8) * 128
    pat_hi = pat_lo + 2 * 4096

    def fire_gathers(h, b):
        pltpu.make_async_copy(idx_hbm.at[h, pl.ds(col0, CW)], idx_v.at[b],
                              isem[b]).wait()
        for j in range(BT_PER_W):
            pltpu.async_copy(
                table_hbm.at[idx_v.at[b, pl.ds(j * 128, 128)]],
                g_v.at[b, pl.ds(j * 128, 128)], gsem[b])

    def drain_gathers(b):
        for j in range(BT_PER_W):
            pltpu.make_async_copy(
                table_hbm.at[idx_v.at[b, pl.ds(j * 128, 128)]],
                g_v.at[b, pl.ds(j * 128, 128)], gsem[b]).wait()

    def transpose(c):
        @plsc.parallel_loop(0, CW // 16, unroll=4)
        def _(rb):
            rows = rb * 16 + iota16
            j = rb >> 3
            base = (rb & 7) * 16 + j * 1024
            for d in range(EMB):
                cols = jnp.full((16,), d, dtype=jnp.int32)
                vec = plsc.load_gather(g_v.at[c], [rows, cols])
                off = base + ((d // 8) * 4096 + (d % 8) * 128)
                t_v[c, pl.ds(off, 16)] = vec

    def store_out(h, c):
        for dt in range(EMB // 8):
            pltpu.async_copy(
                t_v.at[c, pl.ds(dt * 4096, 4096)],
                out_hbm.at[h, pl.ds(dt * (HROW // 4) + bt0 * 1024, 4096)],
                osem[c])

    def wait_store(h, c):
        for dt in range(EMB // 8):
            pltpu.make_async_copy(
                t_v.at[c, pl.ds(dt * 4096, 4096)],
                out_hbm.at[h, pl.ds(dt * (HROW // 4) + bt0 * 1024, 4096)],
                osem[c]).wait()

    pltpu.async_copy(idx_hbm.at[0, pl.ds(col0, CW)], idx_v.at[0], isem[0])

    def outer(t, carry):
        for b in range(2):
            h = t * 2 + b
            c = 1 - b
            fire_gathers(h, b)

            @pl.when(h >= 1)
            def _():
                drain_gathers(c)

            @pl.when(h + 1 < HIST)
            def _():
                pltpu.async_copy(idx_hbm.at[h + 1, pl.ds(col0, CW)],
                                 idx_v.at[c], isem[c])

            @pl.when(h >= 3)
            def _():
                wait_store(h - 3, c)

            @pl.when(h >= 1)
            def _():
                transpose(c)
                store_out(h - 1, c)
        return carry

    lax.fori_loop(0, HIST // 2, outer, 0)

    drain_gathers(1)
    wait_store(HIST - 3, 1)
    transpose(1)
    store_out(HIST - 1, 1)
    wait_store(HIST - 2, 0)
    wait_store(HIST - 1, 1)


def kernel(inputs, table):
    idx_t = inputs.T
    r = _emb_lookup(idx_t, table)
    r = r.reshape(HIST, EMB // 8, BT, 8, 128)
    return r.transpose(2, 4, 0, 1, 3).reshape(BATCH, HIST, EMB)


# speedup vs baseline: 2.3052x; 2.3052x over previous
import functools

import jax
import jax.numpy as jnp
from jax import lax
from jax.experimental import pallas as pl
from jax.experimental.pallas import tpu as pltpu
from jax.experimental.pallas import tpu_sc as plsc

EMB = 32
BATCH = 16384
HIST = 200
NC, NS = 2, 16
NW = NC * NS
BT = BATCH // 128
BT_PER_W = BT // NW
CW = BT_PER_W * 128
TW = BT_PER_W * 128 * EMB
HROW = BT * 128 * EMB

_mesh = plsc.VectorSubcoreMesh(core_axis_name="c", subcore_axis_name="s")


@functools.partial(
    pl.kernel,
    mesh=_mesh,
    out_type=jax.ShapeDtypeStruct((HIST, HROW), jnp.float32),
    scratch_types=[
        pltpu.VMEM((2, CW), jnp.int32),
        pltpu.VMEM((2, CW, EMB), jnp.float32),
        pltpu.VMEM((2, TW), jnp.float32),
        pltpu.SemaphoreType.DMA,
        pltpu.SemaphoreType.DMA,
        pltpu.SemaphoreType.DMA,
        pltpu.SemaphoreType.DMA,
        pltpu.SemaphoreType.DMA,
        pltpu.SemaphoreType.DMA,
    ],
    compiler_params=pltpu.CompilerParams(use_tc_tiling_on_sc=False,
                                         needs_layout_passes=False),
)
def _emb_lookup(idx_hbm, table_hbm, out_hbm, idx_v, g_v, t_v,
                isem0, isem1, gsem0, gsem1, osem0, osem1):
    wid = lax.axis_index("s") * NC + lax.axis_index("c")
    col0 = wid * CW
    bt0 = wid * BT_PER_W
    isem = (isem0, isem1)
    gsem = (gsem0, gsem1)
    osem = (osem0, osem1)
    iota16 = jnp.arange(16, dtype=jnp.int32)
    pat_lo = (iota16 // 8) * 4096 + (iota16 % 8) * 128
    pat_hi = pat_lo + 2 * 4096

    def fire_gathers(h, b):
        pltpu.make_async_copy(idx_hbm.at[h, pl.ds(col0, CW)], idx_v.at[b],
                              isem[b]).wait()
        for j in range(BT_PER_W):
            pltpu.async_copy(
                table_hbm.at[idx_v.at[b, pl.ds(j * 128, 128)]],
                g_v.at[b, pl.ds(j * 128, 128)], gsem[b])

    def drain_gathers(b):
        for j in range(BT_PER_W):
            pltpu.make_async_copy(
                table_hbm.at[idx_v.at[b, pl.ds(j * 128, 128)]],
                g_v.at[b, pl.ds(j * 128, 128)], gsem[b]).wait()

    def transpose(c):
        @plsc.parallel_loop(0, CW // 16, unroll=2)
        def _(rb):
            rows = rb * 16 + iota16
            s = (rb >> 3) * 1024 + (rb & 7) * 16
            for dhalf in range(2):
                for d0 in range(16):
                    m = (d0 + 5 * iota16) & 15
                    cols = dhalf * 16 + m
                    vec = plsc.load_gather(g_v.at[c], [rows, cols])
                    wpat = (dhalf * 8192 + (m // 8) * 4096
                            + (m % 8) * 128 + iota16)
                    plsc.store_scatter(t_v.at[c], [wpat + s], vec)

    def store_out(h, c):
        for dt in range(EMB // 8):
            pltpu.async_copy(
                t_v.at[c, pl.ds(dt * 4096, 4096)],
                out_hbm.at[h, pl.ds(dt * (HROW // 4) + bt0 * 1024, 4096)],
                osem[c])

    def wait_store(h, c):
        for dt in range(EMB // 8):
            pltpu.make_async_copy(
                t_v.at[c, pl.ds(dt * 4096, 4096)],
                out_hbm.at[h, pl.ds(dt * (HROW // 4) + bt0 * 1024, 4096)],
                osem[c]).wait()

    pltpu.async_copy(idx_hbm.at[0, pl.ds(col0, CW)], idx_v.at[0], isem[0])

    def outer(t, carry):
        for b in range(2):
            h = t * 2 + b
            c = 1 - b
            fire_gathers(h, b)

            @pl.when(h >= 1)
            def _():
                drain_gathers(c)

            @pl.when(h + 1 < HIST)
            def _():
                pltpu.async_copy(idx_hbm.at[h + 1, pl.ds(col0, CW)],
                                 idx_v.at[c], isem[c])

            @pl.when(h >= 3)
            def _():
                wait_store(h - 3, c)

            @pl.when(h >= 1)
            def _():
                transpose(c)
                store_out(h - 1, c)
        return carry

    lax.fori_loop(0, HIST // 2, outer, 0)

    drain_gathers(1)
    wait_store(HIST - 3, 1)
    transpose(1)
    store_out(HIST - 1, 1)
    wait_store(HIST - 2, 0)
    wait_store(HIST - 1, 1)


def kernel(inputs, table):
    idx_t = inputs.T
    r = _emb_lookup(idx_t, table)
    r = r.reshape(HIST, EMB // 8, BT, 8, 128)
    return r.transpose(2, 4, 0, 1, 3).reshape(BATCH, HIST, EMB)
